# baseline (speedup 1.0000x reference)
import math

import jax
import jax.numpy as jnp
from jax.experimental import pallas as pl
from jax.experimental.pallas import tpu as pltpu

N = 1000
D = 12544
HID = 1024
C = 91
NC = C - 1
SCORE_THRESH = 0.05
NMS_THRESH = 0.5
DETS = 100
IMG_W = 800.0
IMG_H = 800.0
BBOX_XFORM_CLIP = float(math.log(1000.0 / 16.0))

KBLK = 896
KSTEPS = D // KBLK


def _roi_head_kernel(x_ref, w1_ref, prop_ref, b1_ref, w2_ref, b2_ref,
                     wcls_ref, bcls_ref,
                     wdx_ref, wdy_ref, wdw_ref, wdh_ref,
                     bdx_ref, bdy_ref, bdw_ref, bdh_ref,
                     detb_ref, dets_ref, detl_ref,
                     h1_ref, s_ref, x1o_ref, y1o_ref, x2o_ref, y2o_ref,
                     area_ref, rm_ref, rci_ref):
    k = pl.program_id(0)
    part = jnp.dot(x_ref[...], w1_ref[...], preferred_element_type=jnp.float32)

    @pl.when(k == 0)
    def _():
        h1_ref[...] = part

    @pl.when(k > 0)
    def _():
        h1_ref[...] = h1_ref[...] + part

    @pl.when(k == KSTEPS - 1)
    def _():
        h1 = jnp.maximum(h1_ref[...] + b1_ref[...], 0.0)
        h2 = jnp.maximum(
            jnp.dot(h1, w2_ref[...], preferred_element_type=jnp.float32)
            + b2_ref[...], 0.0)
        logits = jnp.dot(h2, wcls_ref[...],
                         preferred_element_type=jnp.float32) + bcls_ref[...]
        dx = jnp.dot(h2, wdx_ref[...],
                     preferred_element_type=jnp.float32) + bdx_ref[...]
        dy = jnp.dot(h2, wdy_ref[...],
                     preferred_element_type=jnp.float32) + bdy_ref[...]
        dw = jnp.dot(h2, wdw_ref[...],
                     preferred_element_type=jnp.float32) + bdw_ref[...]
        dh = jnp.dot(h2, wdh_ref[...],
                     preferred_element_type=jnp.float32) + bdh_ref[...]

        p = prop_ref[...]
        widths = p[:, 2:3] - p[:, 0:1]
        heights = p[:, 3:4] - p[:, 1:2]
        ctr_x = p[:, 0:1] + 0.5 * widths
        ctr_y = p[:, 1:2] + 0.5 * heights
        dx = dx / 10.0
        dy = dy / 10.0
        dw = jnp.minimum(dw / 5.0, BBOX_XFORM_CLIP)
        dh = jnp.minimum(dh / 5.0, BBOX_XFORM_CLIP)
        pred_ctr_x = dx * widths + ctr_x
        pred_ctr_y = dy * heights + ctr_y
        pred_w = jnp.exp(dw) * widths
        pred_h = jnp.exp(dh) * heights
        x1 = jnp.clip(pred_ctr_x - 0.5 * pred_w, 0.0, IMG_W)
        y1 = jnp.clip(pred_ctr_y - 0.5 * pred_h, 0.0, IMG_H)
        x2 = jnp.clip(pred_ctr_x + 0.5 * pred_w, 0.0, IMG_W)
        y2 = jnp.clip(pred_ctr_y + 0.5 * pred_h, 0.0, IMG_H)

        scores = jax.nn.softmax(logits, axis=-1)[:, 1:]
        ws = x2 - x1
        hs = y2 - y1
        valid = (scores > SCORE_THRESH) & (ws >= 0.01) & (hs >= 0.01)
        s2d = jnp.where(valid, scores, -1e9)

        st = jnp.swapaxes(s2d, 0, 1)
        x1t = jnp.swapaxes(x1, 0, 1)
        y1t = jnp.swapaxes(y1, 0, 1)
        x2t = jnp.swapaxes(x2, 0, 1)
        y2t = jnp.swapaxes(y2, 0, 1)
        rowc = jax.lax.broadcasted_iota(jnp.int32, (NC, N), 0).astype(
            jnp.float32)
        lanen = jax.lax.broadcasted_iota(jnp.int32, (NC, N), 1).astype(
            jnp.float32)
        offT = (rowc + 1.0) * (IMG_W + 1.0)
        x1o = x1t + offT
        y1o = y1t + offT
        x2o = x2t + offT
        y2o = y2t + offT
        s_ref[...] = st
        x1o_ref[...] = x1o
        y1o_ref[...] = y1o
        x2o_ref[...] = x2o
        y2o_ref[...] = y2o
        area_ref[...] = (x2o - x1o) * (y2o - y1o)

        idxmat = lanen * float(NC) + rowc
        BIG = 1e9
        rm0 = jnp.max(st, axis=1, keepdims=True)
        rci0 = jnp.min(jnp.where(st == rm0, idxmat, BIG), axis=1,
                       keepdims=True)
        rm_ref[...] = rm0
        rci_ref[...] = rci0

        ci4 = jax.lax.broadcasted_iota(jnp.int32, (1, 4), 1)
        lane1k = jax.lax.broadcasted_iota(jnp.int32, (1, N), 1)
        lane1kf = lane1k.astype(jnp.float32)

        def body(i, carry):
            rm = rm_ref[...]
            m = jnp.max(rm)
            sel = jnp.min(jnp.where(rm == m, rci_ref[...], BIG))
            seli = sel.astype(jnp.int32)
            crow = jax.lax.rem(seli, NC)
            nlane = seli // NC

            xr1 = x1o_ref[pl.ds(crow, 1), :]
            yr1 = y1o_ref[pl.ds(crow, 1), :]
            xr2 = x2o_ref[pl.ds(crow, 1), :]
            yr2 = y2o_ref[pl.ds(crow, 1), :]
            eql = lane1k == nlane
            zero = jnp.zeros((), jnp.float32)
            bx1 = jnp.sum(jnp.where(eql, xr1, zero))
            by1 = jnp.sum(jnp.where(eql, yr1, zero))
            bx2 = jnp.sum(jnp.where(eql, xr2, zero))
            by2 = jnp.sum(jnp.where(eql, yr2, zero))
            ba = (bx2 - bx1) * (by2 - by1)
            bl_i = crow + 1
            boff = bl_i.astype(jnp.float32) * (IMG_W + 1.0)

            rowvals = jnp.where(
                ci4 == 0, bx1 - boff,
                jnp.where(ci4 == 1, by1 - boff,
                          jnp.where(ci4 == 2, bx2 - boff, by2 - boff)))
            detb_ref[pl.ds(i, 1), :] = rowvals
            dets_ref[pl.ds(i, 1), :] = (jnp.zeros((1, 1), jnp.float32)
                                        + jnp.maximum(m, 0.0))
            detl_ref[pl.ds(i, 1), :] = jnp.zeros((1, 1), jnp.int32) + bl_i

            sr = s_ref[pl.ds(crow, 1), :]
            ar = area_ref[pl.ds(crow, 1), :]
            ltx = jnp.maximum(bx1, xr1)
            lty = jnp.maximum(by1, yr1)
            rbx = jnp.minimum(bx2, xr2)
            rby = jnp.minimum(by2, yr2)
            iw = jnp.maximum(rbx - ltx, 0.0)
            ih = jnp.maximum(rby - lty, 0.0)
            inter = iw * ih
            iou = inter / (ba + ar - inter + 1e-9)
            s_new = jnp.where(iou > NMS_THRESH, -1e9, sr)
            s_ref[pl.ds(crow, 1), :] = s_new
            rm_c = jnp.max(s_new)
            idxrow = lane1kf * float(NC) + crow.astype(jnp.float32)
            rci_c = jnp.min(jnp.where(s_new == rm_c, idxrow, BIG))
            rm_ref[pl.ds(crow, 1), :] = jnp.zeros((1, 1), jnp.float32) + rm_c
            rci_ref[pl.ds(crow, 1), :] = (jnp.zeros((1, 1), jnp.float32)
                                          + rci_c)
            return carry

        jax.lax.fori_loop(0, DETS, body, 0)


def kernel(x, proposals, w1, b1, w2, b2, w_cls, b_cls, w_bbox, b_bbox):
    wb = w_bbox.reshape(HID, C, 4)[:, 1:, :]
    bb = b_bbox.reshape(C, 4)[1:, :]
    wdx, wdy, wdw, wdh = (wb[:, :, j] for j in range(4))
    bdx, bdy, bdw, bdh = (bb[:, j].reshape(1, NC) for j in range(4))

    full = lambda shape: pl.BlockSpec(shape, lambda k: (0, 0))
    detb, dets, detl = pl.pallas_call(
        _roi_head_kernel,
        grid=(KSTEPS,),
        in_specs=[
            pl.BlockSpec((N, KBLK), lambda k: (0, k)),
            pl.BlockSpec((KBLK, HID), lambda k: (k, 0)),
            full((N, 4)),
            full((1, HID)),
            full((HID, HID)),
            full((1, HID)),
            full((HID, C)),
            full((1, C)),
            full((HID, NC)), full((HID, NC)), full((HID, NC)), full((HID, NC)),
            full((1, NC)), full((1, NC)), full((1, NC)), full((1, NC)),
        ],
        out_specs=[full((DETS, 4)), full((DETS, 1)), full((DETS, 1))],
        out_shape=[
            jax.ShapeDtypeStruct((DETS, 4), jnp.float32),
            jax.ShapeDtypeStruct((DETS, 1), jnp.float32),
            jax.ShapeDtypeStruct((DETS, 1), jnp.int32),
        ],
        scratch_shapes=[
            pltpu.VMEM((N, HID), jnp.float32),
            pltpu.VMEM((NC, N), jnp.float32),
            pltpu.VMEM((NC, N), jnp.float32),
            pltpu.VMEM((NC, N), jnp.float32),
            pltpu.VMEM((NC, N), jnp.float32),
            pltpu.VMEM((NC, N), jnp.float32),
            pltpu.VMEM((NC, N), jnp.float32),
            pltpu.VMEM((NC, 1), jnp.float32),
            pltpu.VMEM((NC, 1), jnp.float32),
        ],
        compiler_params=pltpu.CompilerParams(
            dimension_semantics=("arbitrary",)),
    )(x, w1, proposals, b1.reshape(1, HID), w2, b2.reshape(1, HID),
      w_cls, b_cls.reshape(1, C), wdx, wdy, wdw, wdh, bdx, bdy, bdw, bdh)
    return detb, dets.reshape(DETS), detl.reshape(DETS)


# speedup vs baseline: 1.0119x; 1.0119x over previous
import math

import jax
import jax.numpy as jnp
from jax.experimental import pallas as pl
from jax.experimental.pallas import tpu as pltpu

N = 1000
D = 12544
HID = 1024
C = 91
NC = C - 1
SCORE_THRESH = 0.05
NMS_THRESH = 0.5
DETS = 100
IMG_W = 800.0
IMG_H = 800.0
BBOX_XFORM_CLIP = float(math.log(1000.0 / 16.0))

KBLK = 896
KSTEPS = D // KBLK


def _roi_head_kernel(x_ref, w1_ref, prop_ref, b1_ref, w2_ref, b2_ref,
                     wcls_ref, bcls_ref,
                     wdx_ref, wdy_ref, wdw_ref, wdh_ref,
                     bdx_ref, bdy_ref, bdw_ref, bdh_ref,
                     detb_ref, dets_ref, detl_ref,
                     h1_ref, s_ref, x1o_ref, y1o_ref, x2o_ref, y2o_ref,
                     area_ref, rm_ref, rci_ref):
    k = pl.program_id(0)
    part = jnp.dot(x_ref[...], w1_ref[...], preferred_element_type=jnp.float32)

    @pl.when(k == 0)
    def _():
        h1_ref[...] = part

    @pl.when(k > 0)
    def _():
        h1_ref[...] = h1_ref[...] + part

    @pl.when(k == KSTEPS - 1)
    def _():
        h1 = jnp.maximum(h1_ref[...] + b1_ref[...], 0.0)
        h2 = jnp.maximum(
            jnp.dot(h1, w2_ref[...], preferred_element_type=jnp.float32)
            + b2_ref[...], 0.0)
        logits = jnp.dot(h2, wcls_ref[...],
                         preferred_element_type=jnp.float32) + bcls_ref[...]
        dx = jnp.dot(h2, wdx_ref[...],
                     preferred_element_type=jnp.float32) + bdx_ref[...]
        dy = jnp.dot(h2, wdy_ref[...],
                     preferred_element_type=jnp.float32) + bdy_ref[...]
        dw = jnp.dot(h2, wdw_ref[...],
                     preferred_element_type=jnp.float32) + bdw_ref[...]
        dh = jnp.dot(h2, wdh_ref[...],
                     preferred_element_type=jnp.float32) + bdh_ref[...]

        p = prop_ref[...]
        widths = p[:, 2:3] - p[:, 0:1]
        heights = p[:, 3:4] - p[:, 1:2]
        ctr_x = p[:, 0:1] + 0.5 * widths
        ctr_y = p[:, 1:2] + 0.5 * heights
        dx = dx / 10.0
        dy = dy / 10.0
        dw = jnp.minimum(dw / 5.0, BBOX_XFORM_CLIP)
        dh = jnp.minimum(dh / 5.0, BBOX_XFORM_CLIP)
        pred_ctr_x = dx * widths + ctr_x
        pred_ctr_y = dy * heights + ctr_y
        pred_w = jnp.exp(dw) * widths
        pred_h = jnp.exp(dh) * heights
        x1 = jnp.clip(pred_ctr_x - 0.5 * pred_w, 0.0, IMG_W)
        y1 = jnp.clip(pred_ctr_y - 0.5 * pred_h, 0.0, IMG_H)
        x2 = jnp.clip(pred_ctr_x + 0.5 * pred_w, 0.0, IMG_W)
        y2 = jnp.clip(pred_ctr_y + 0.5 * pred_h, 0.0, IMG_H)

        scores = jax.nn.softmax(logits, axis=-1)[:, 1:]
        ws = x2 - x1
        hs = y2 - y1
        valid = (scores > SCORE_THRESH) & (ws >= 0.01) & (hs >= 0.01)
        s2d = jnp.where(valid, scores, -1e9)

        st = jnp.swapaxes(s2d, 0, 1)
        x1t = jnp.swapaxes(x1, 0, 1)
        y1t = jnp.swapaxes(y1, 0, 1)
        x2t = jnp.swapaxes(x2, 0, 1)
        y2t = jnp.swapaxes(y2, 0, 1)
        rowc = jax.lax.broadcasted_iota(jnp.int32, (NC, N), 0).astype(
            jnp.float32)
        lanen = jax.lax.broadcasted_iota(jnp.int32, (NC, N), 1).astype(
            jnp.float32)
        offT = (rowc + 1.0) * (IMG_W + 1.0)
        x1o = x1t + offT
        y1o = y1t + offT
        x2o = x2t + offT
        y2o = y2t + offT
        s_ref[...] = st
        x1o_ref[...] = x1o
        y1o_ref[...] = y1o
        x2o_ref[...] = x2o
        y2o_ref[...] = y2o
        area_ref[...] = (x2o - x1o) * (y2o - y1o)

        idxmat = lanen * float(NC) + rowc
        BIG = 1e9
        rm0 = jnp.max(st, axis=1, keepdims=True)
        rci0 = jnp.min(jnp.where(st == rm0, idxmat, BIG), axis=1,
                       keepdims=True)
        rm_ref[...] = jnp.swapaxes(rm0, 0, 1)
        rci_ref[...] = jnp.swapaxes(rci0, 0, 1)

        ci4 = jax.lax.broadcasted_iota(jnp.int32, (1, 4), 1)
        lane1k = jax.lax.broadcasted_iota(jnp.int32, (1, N), 1)
        lane1kf = lane1k.astype(jnp.float32)
        lane90 = jax.lax.broadcasted_iota(jnp.int32, (1, NC), 1)

        def body(i, carry):
            rm = rm_ref[...]
            m = jnp.max(rm)
            sel = jnp.min(jnp.where(rm == m, rci_ref[...], BIG))
            seli = sel.astype(jnp.int32)
            crow = jax.lax.rem(seli, NC)
            nlane = seli // NC

            xr1 = x1o_ref[pl.ds(crow, 1), :]
            yr1 = y1o_ref[pl.ds(crow, 1), :]
            xr2 = x2o_ref[pl.ds(crow, 1), :]
            yr2 = y2o_ref[pl.ds(crow, 1), :]
            eql = lane1k == nlane
            zero = jnp.zeros((), jnp.float32)
            bx1 = jnp.sum(jnp.where(eql, xr1, zero))
            by1 = jnp.sum(jnp.where(eql, yr1, zero))
            bx2 = jnp.sum(jnp.where(eql, xr2, zero))
            by2 = jnp.sum(jnp.where(eql, yr2, zero))
            ba = (bx2 - bx1) * (by2 - by1)
            bl_i = crow + 1
            boff = bl_i.astype(jnp.float32) * (IMG_W + 1.0)

            rowvals = jnp.where(
                ci4 == 0, bx1 - boff,
                jnp.where(ci4 == 1, by1 - boff,
                          jnp.where(ci4 == 2, bx2 - boff, by2 - boff)))
            detb_ref[pl.ds(i, 1), :] = rowvals
            dets_ref[pl.ds(i, 1), :] = (jnp.zeros((1, 1), jnp.float32)
                                        + jnp.maximum(m, 0.0))
            detl_ref[pl.ds(i, 1), :] = jnp.zeros((1, 1), jnp.int32) + bl_i

            sr = s_ref[pl.ds(crow, 1), :]
            ar = area_ref[pl.ds(crow, 1), :]
            ltx = jnp.maximum(bx1, xr1)
            lty = jnp.maximum(by1, yr1)
            rbx = jnp.minimum(bx2, xr2)
            rby = jnp.minimum(by2, yr2)
            iw = jnp.maximum(rbx - ltx, 0.0)
            ih = jnp.maximum(rby - lty, 0.0)
            inter = iw * ih
            iou = inter / (ba + ar - inter + 1e-9)
            s_new = jnp.where(iou > NMS_THRESH, -1e9, sr)
            s_ref[pl.ds(crow, 1), :] = s_new
            rm_c = jnp.max(s_new)
            idxrow = lane1kf * float(NC) + crow.astype(jnp.float32)
            rci_c = jnp.min(jnp.where(s_new == rm_c, idxrow, BIG))
            sel_lane = lane90 == crow
            rm_ref[...] = jnp.where(sel_lane, rm_c, rm)
            rci_ref[...] = jnp.where(sel_lane, rci_c, rci_ref[...])
            return carry

        jax.lax.fori_loop(0, DETS, body, 0)


def kernel(x, proposals, w1, b1, w2, b2, w_cls, b_cls, w_bbox, b_bbox):
    wb = w_bbox.reshape(HID, C, 4)[:, 1:, :]
    bb = b_bbox.reshape(C, 4)[1:, :]
    wdx, wdy, wdw, wdh = (wb[:, :, j] for j in range(4))
    bdx, bdy, bdw, bdh = (bb[:, j].reshape(1, NC) for j in range(4))

    full = lambda shape: pl.BlockSpec(shape, lambda k: (0, 0))
    detb, dets, detl = pl.pallas_call(
        _roi_head_kernel,
        grid=(KSTEPS,),
        in_specs=[
            pl.BlockSpec((N, KBLK), lambda k: (0, k)),
            pl.BlockSpec((KBLK, HID), lambda k: (k, 0)),
            full((N, 4)),
            full((1, HID)),
            full((HID, HID)),
            full((1, HID)),
            full((HID, C)),
            full((1, C)),
            full((HID, NC)), full((HID, NC)), full((HID, NC)), full((HID, NC)),
            full((1, NC)), full((1, NC)), full((1, NC)), full((1, NC)),
        ],
        out_specs=[full((DETS, 4)), full((DETS, 1)), full((DETS, 1))],
        out_shape=[
            jax.ShapeDtypeStruct((DETS, 4), jnp.float32),
            jax.ShapeDtypeStruct((DETS, 1), jnp.float32),
            jax.ShapeDtypeStruct((DETS, 1), jnp.int32),
        ],
        scratch_shapes=[
            pltpu.VMEM((N, HID), jnp.float32),
            pltpu.VMEM((NC, N), jnp.float32),
            pltpu.VMEM((NC, N), jnp.float32),
            pltpu.VMEM((NC, N), jnp.float32),
            pltpu.VMEM((NC, N), jnp.float32),
            pltpu.VMEM((NC, N), jnp.float32),
            pltpu.VMEM((NC, N), jnp.float32),
            pltpu.VMEM((1, NC), jnp.float32),
            pltpu.VMEM((1, NC), jnp.float32),
        ],
        compiler_params=pltpu.CompilerParams(
            dimension_semantics=("arbitrary",)),
    )(x, w1, proposals, b1.reshape(1, HID), w2, b2.reshape(1, HID),
      w_cls, b_cls.reshape(1, C), wdx, wdy, wdw, wdh, bdx, bdy, bdw, bdh)
    return detb, dets.reshape(DETS), detl.reshape(DETS)
